# baseline (speedup 1.0000x reference)
import functools

import jax
import jax.numpy as jnp
from jax import lax
from jax.experimental import pallas as pl
from jax.experimental.pallas import tpu as pltpu
from jax.experimental.pallas import tpu_sc as plsc

_BATCH = 4096
_SEQ_LEN = 200
_EMSIZE = 64
_VOCAB = 1000000

_NC = 2
_NS = 16
_NW = _NC * _NS

_ROWS_PER_W = _BATCH // _NW
_CHUNKS = (104, 96)
_OFFS = (0, 104)
_K = 2
_SLOTS = 4
_NGROUP = _ROWS_PER_W

_mesh = plsc.VectorSubcoreMesh(core_axis_name="c", subcore_axis_name="s")


@functools.partial(
    pl.kernel,
    out_type=jax.ShapeDtypeStruct((_BATCH * _SEQ_LEN, 2 * _EMSIZE), jnp.float32),
    mesh=_mesh,
    scratch_types=[
        pltpu.VMEM((_ROWS_PER_W, _SEQ_LEN), jnp.int32),
        pltpu.VMEM((_SLOTS, _CHUNKS[0], 2 * _EMSIZE), jnp.float32),
        pltpu.VMEM((_SLOTS, _CHUNKS[1], 2 * _EMSIZE), jnp.float32),
        pltpu.SemaphoreType.DMA((_SLOTS,)),
        pltpu.SemaphoreType.DMA((_SLOTS,)),
    ],
    compiler_params=pltpu.CompilerParams(use_tc_tiling_on_sc=False),
)
def _embed_sc(seq_hbm, table_hbm, out_hbm, idx_v, rows_a, rows_b, gsem, wsem):
    wid = lax.axis_index("s") * _NC + lax.axis_index("c")
    b0 = wid * _ROWS_PER_W
    pltpu.sync_copy(seq_hbm.at[pl.ds(b0, _ROWS_PER_W), :], idx_v)

    _rings = (rows_a, rows_b)

    def _issue_gathers(g, s):
        for k in range(_K):
            pltpu.async_copy(
                table_hbm.at[idx_v.at[g, pl.ds(_OFFS[k], _CHUNKS[k])]],
                _rings[k].at[s],
                gsem.at[s],
            )

    def _drain_gathers(g, s):
        for k in range(_K):
            pltpu.make_async_copy(
                table_hbm.at[idx_v.at[g, pl.ds(_OFFS[k], _CHUNKS[k])]],
                _rings[k].at[s],
                gsem.at[s],
            ).wait()

    def _writes(g, s, wait):
        for k in range(_K):
            row0 = (b0 + g) * _SEQ_LEN + _OFFS[k]
            cp = pltpu.make_async_copy(
                _rings[k].at[s],
                out_hbm.at[pl.ds(row0, _CHUNKS[k]), :],
                wsem.at[s],
            )
            if wait:
                cp.wait()
            else:
                cp.start()

    def _body(g, s, drain_old, issue_ahead):
        _drain_gathers(g, s)
        _writes(g, s, wait=False)
        if drain_old:
            _writes(g - 2, (s + 2) % _SLOTS, wait=True)
        if issue_ahead:
            _issue_gathers(g + 2, (s + 2) % _SLOTS)

    _issue_gathers(0, 0)
    _issue_gathers(1, 1)
    _body(0, 0, False, True)
    _body(1, 1, False, True)

    @pl.loop(0, (_NGROUP - 4) // _SLOTS)
    def _steady(t):
        g0 = 2 + t * _SLOTS
        for q in range(_SLOTS):
            _body(g0 + q, (2 + q) % _SLOTS, True, True)

    _body(_NGROUP - 2, (_NGROUP - 2) % _SLOTS, True, False)
    _body(_NGROUP - 1, (_NGROUP - 1) % _SLOTS, True, False)
    _writes(_NGROUP - 2, (_NGROUP - 2) % _SLOTS, wait=True)
    _writes(_NGROUP - 1, (_NGROUP - 1) % _SLOTS, wait=True)


def kernel(sequence, src_word_table):
    tp = jnp.pad(src_word_table, ((0, 0), (0, _EMSIZE)))
    out = _embed_sc(sequence, tp)
    return jnp.reshape(out, (_BATCH, _SEQ_LEN, 2 * _EMSIZE))[:, :, :_EMSIZE]


# speedup vs baseline: 1.0904x; 1.0904x over previous
import functools

import jax
import jax.numpy as jnp
from jax import lax
from jax.experimental import pallas as pl
from jax.experimental.pallas import tpu as pltpu
from jax.experimental.pallas import tpu_sc as plsc

_BATCH = 4096
_SEQ_LEN = 200
_EMSIZE = 64
_VOCAB = 1000000

_NC = 2
_NS = 16
_NW = _NC * _NS

_ROWS_PER_W = _BATCH // _NW
_CHUNKS = (128, 72)
_OFFS = (0, 128)
_K = 2
_SLOTS = 4
_NGROUP = _ROWS_PER_W

_mesh = plsc.VectorSubcoreMesh(core_axis_name="c", subcore_axis_name="s")


@functools.partial(
    pl.kernel,
    out_type=jax.ShapeDtypeStruct((_BATCH * _SEQ_LEN, 2 * _EMSIZE), jnp.float32),
    mesh=_mesh,
    scratch_types=[
        pltpu.VMEM((_ROWS_PER_W, _SEQ_LEN), jnp.int32),
        pltpu.VMEM((_SLOTS, _CHUNKS[0], _EMSIZE), jnp.float32),
        pltpu.VMEM((_SLOTS, _CHUNKS[1], _EMSIZE), jnp.float32),
        pltpu.SemaphoreType.DMA((_SLOTS,)),
        pltpu.SemaphoreType.DMA((_SLOTS,)),
    ],
    compiler_params=pltpu.CompilerParams(use_tc_tiling_on_sc=False),
)
def _embed_sc(seq_hbm, table_hbm, out_hbm, idx_v, rows_a, rows_b, gsem, wsem):
    wid = lax.axis_index("s") * _NC + lax.axis_index("c")
    b0 = wid * _ROWS_PER_W
    pltpu.sync_copy(seq_hbm.at[pl.ds(b0, _ROWS_PER_W), :], idx_v)

    _rings = (rows_a, rows_b)

    def _issue_gathers(g, s):
        for k in range(_K):
            pltpu.async_copy(
                table_hbm.at[idx_v.at[g, pl.ds(_OFFS[k], _CHUNKS[k])]],
                _rings[k].at[s],
                gsem.at[s],
            )

    def _drain_gathers(g, s):
        for k in range(_K):
            pltpu.make_async_copy(
                table_hbm.at[idx_v.at[g, pl.ds(_OFFS[k], _CHUNKS[k])]],
                _rings[k].at[s],
                gsem.at[s],
            ).wait()

    def _writes(g, s, wait):
        for k in range(_K):
            row0 = (b0 + g) * _SEQ_LEN + _OFFS[k]
            cp = pltpu.make_async_copy(
                _rings[k].at[s],
                out_hbm.at[pl.ds(row0, _CHUNKS[k]), pl.ds(0, _EMSIZE)],
                wsem.at[s],
            )
            if wait:
                cp.wait()
            else:
                cp.start()

    def _body(g, s, drain_old, issue_ahead):
        _drain_gathers(g, s)
        _writes(g, s, wait=False)
        if drain_old:
            _writes(g - 2, (s + 2) % _SLOTS, wait=True)
        if issue_ahead:
            _issue_gathers(g + 2, (s + 2) % _SLOTS)

    _issue_gathers(0, 0)
    _issue_gathers(1, 1)
    _body(0, 0, False, True)
    _body(1, 1, False, True)

    @pl.loop(0, (_NGROUP - 4) // _SLOTS)
    def _steady(t):
        g0 = 2 + t * _SLOTS
        for q in range(_SLOTS):
            _body(g0 + q, (2 + q) % _SLOTS, True, True)

    _body(_NGROUP - 2, (_NGROUP - 2) % _SLOTS, True, False)
    _body(_NGROUP - 1, (_NGROUP - 1) % _SLOTS, True, False)
    _writes(_NGROUP - 2, (_NGROUP - 2) % _SLOTS, wait=True)
    _writes(_NGROUP - 1, (_NGROUP - 1) % _SLOTS, wait=True)


def kernel(sequence, src_word_table):
    out = _embed_sc(sequence, src_word_table)
    return jnp.reshape(out, (_BATCH, _SEQ_LEN, 2 * _EMSIZE))[:, :, :_EMSIZE]
